# baseline (speedup 1.0000x reference)
import functools

import jax
import jax.numpy as jnp
from jax import lax
from jax.experimental import pallas as pl
from jax.experimental.pallas import tpu as pltpu
from jax.experimental.pallas import tpu_sc as plsc

_NUM_CLASSES = 10
_FEAT_DIM = 2
_ALPHA = 0.5
_BATCH = 16384
_L = 16
_NW = 16
_SPW = _BATCH // _NW
_CH = _SPW // _L
_TAB = _NUM_CLASSES * _L
_CLEN = _NUM_CLASSES * _FEAT_DIM


def _sc_body(f_h, lab_h, c_h,
             res_h, nc_h,
             f_v, lab_v, res_v, c_v, cx_v, cy_v,
             cnt_t, sx_t, sy_t, part_v, rb_v, nc_v,
             shared):
    sid = lax.axis_index("s")
    base = sid * _SPW

    pltpu.sync_copy(f_h.at[pl.ds(base * _FEAT_DIM, _SPW * _FEAT_DIM)], f_v)
    pltpu.sync_copy(lab_h.at[pl.ds(base, _SPW)], lab_v)
    pltpu.sync_copy(c_h, c_v)

    lane = lax.iota(jnp.int32, _L)
    lane2 = lane * 2
    cmask = lane < _NUM_CLASSES
    cx_v[...] = plsc.load_gather(c_v, [lane2], mask=cmask)
    cy_v[...] = plsc.load_gather(c_v, [lane2 + 1], mask=cmask)

    zero = jnp.zeros((_L,), jnp.float32)
    for c in range(_NUM_CLASSES):
        sl = pl.ds(c * _L, _L)
        cnt_t[sl] = zero
        sx_t[sl] = zero
        sy_t[sl] = zero

    ones = jnp.ones((_L,), jnp.float32)

    for i in range(_CH):
        sl = pl.ds(i * _L, _L)
        lab = lab_v[sl]
        fidx = lane2 + (2 * _L) * i
        fxc = plsc.load_gather(f_v, [fidx])
        fyc = plsc.load_gather(f_v, [fidx + 1])
        cxg = plsc.load_gather(cx_v, [lab])
        cyg = plsc.load_gather(cy_v, [lab])
        dx = fxc - cxg
        dy = fyc - cyg
        res_v[sl] = dx * dx + dy * dy
        slot = lab * _L + lane
        plsc.addupdate_scatter(cnt_t, [slot], ones)
        plsc.addupdate_scatter(sx_t, [slot], fxc)
        plsc.addupdate_scatter(sy_t, [slot], fyc)

    pltpu.sync_copy(res_v, res_h.at[pl.ds(base, _SPW)])

    cntv = zero
    sxv = zero
    syv = zero
    for c in range(_NUM_CLASSES):
        m = lane == c
        sl = pl.ds(c * _L, _L)
        cntv = jnp.where(m, jnp.sum(cnt_t[sl]), cntv)
        sxv = jnp.where(m, jnp.sum(sx_t[sl]), sxv)
        syv = jnp.where(m, jnp.sum(sy_t[sl]), syv)
    part_v[pl.ds(0, _L)] = cntv
    part_v[pl.ds(_L, _L)] = sxv
    part_v[pl.ds(2 * _L, _L)] = syv
    pltpu.sync_copy(part_v, shared.at[pl.ds(sid * 3 * _L, 3 * _L)])

    plsc.subcore_barrier()

    @pl.when(sid == 0)
    def _finalize():
        pltpu.sync_copy(shared, rb_v)
        cnt = jnp.zeros((_L,), jnp.float32)
        sx = jnp.zeros((_L,), jnp.float32)
        sy = jnp.zeros((_L,), jnp.float32)
        for w in range(_NW):
            cnt = cnt + rb_v[pl.ds(w * 3 * _L, _L)]
            sx = sx + rb_v[pl.ds(w * 3 * _L + _L, _L)]
            sy = sy + rb_v[pl.ds(w * 3 * _L + 2 * _L, _L)]
        cxr = cx_v[...]
        cyr = cy_v[...]
        scale = _ALPHA / (1.0 + cnt)
        plsc.store_scatter(nc_v, [lane2],
                           cxr - scale * (cnt * cxr - sx), mask=cmask)
        plsc.store_scatter(nc_v, [lane2 + 1],
                           cyr - scale * (cnt * cyr - sy), mask=cmask)
        pltpu.sync_copy(nc_v, nc_h)


_sc_call = functools.partial(
    pl.kernel,
    out_type=(
        jax.ShapeDtypeStruct((_BATCH,), jnp.float32),
        jax.ShapeDtypeStruct((_CLEN,), jnp.float32),
    ),
    mesh=plsc.VectorSubcoreMesh(
        core_axis_name="c", subcore_axis_name="s", num_cores=1
    ),
    compiler_params=pltpu.CompilerParams(needs_layout_passes=False),
    scratch_types=(
        pltpu.VMEM((_SPW * _FEAT_DIM,), jnp.float32),
        pltpu.VMEM((_SPW,), jnp.int32),
        pltpu.VMEM((_SPW,), jnp.float32),
        pltpu.VMEM((_CLEN,), jnp.float32),
        pltpu.VMEM((_L,), jnp.float32),
        pltpu.VMEM((_L,), jnp.float32),
        pltpu.VMEM((_TAB,), jnp.float32),
        pltpu.VMEM((_TAB,), jnp.float32),
        pltpu.VMEM((_TAB,), jnp.float32),
        pltpu.VMEM((3 * _L,), jnp.float32),
        pltpu.VMEM((_NW * 3 * _L,), jnp.float32),
        pltpu.VMEM((_CLEN,), jnp.float32),
        pltpu.VMEM_SHARED((_NW * 3 * _L,), jnp.float32),
    ),
)(_sc_body)


def kernel(features, labels, centers):
    res, nc = _sc_call(
        features.reshape(-1), labels.reshape(-1), centers.reshape(-1)
    )
    return (res.reshape(_BATCH, 1), nc.reshape(_NUM_CLASSES, _FEAT_DIM))


# speedup vs baseline: 1.4165x; 1.4165x over previous
import functools

import jax
import jax.numpy as jnp
from jax import lax
from jax.experimental import pallas as pl
from jax.experimental.pallas import tpu as pltpu
from jax.experimental.pallas import tpu_sc as plsc

_NUM_CLASSES = 10
_FEAT_DIM = 2
_ALPHA = 0.5
_BATCH = 16384
_L = 16
_NW = 16
_SPW = _BATCH // _NW
_CH = _SPW // _L
_TAB = _NUM_CLASSES * _L
_CLEN = _NUM_CLASSES * _FEAT_DIM


def _sc_body(fx_h, fy_h, lab_h, c_h,
             res_h, nc_h,
             fx_v, fy_v, lab_v, res_v, c_v, cx_v, cy_v,
             cnt_t, sx_t, sy_t, part_v, rb_v, nc_v,
             shared,
             sem):
    sid = lax.axis_index("s")
    base = sid * _SPW

    cp1 = pltpu.async_copy(fx_h.at[pl.ds(base, _SPW)], fx_v, sem)
    cp2 = pltpu.async_copy(fy_h.at[pl.ds(base, _SPW)], fy_v, sem)
    cp3 = pltpu.async_copy(lab_h.at[pl.ds(base, _SPW)], lab_v, sem)
    pltpu.sync_copy(c_h, c_v)

    lane = lax.iota(jnp.int32, _L)
    lane2 = lane * 2
    cmask = lane < _NUM_CLASSES
    cx_v[...] = plsc.load_gather(c_v, [lane2], mask=cmask)
    cy_v[...] = plsc.load_gather(c_v, [lane2 + 1], mask=cmask)

    zero = jnp.zeros((_L,), jnp.float32)
    for c in range(_NUM_CLASSES):
        sl = pl.ds(c * _L, _L)
        cnt_t[sl] = zero
        sx_t[sl] = zero
        sy_t[sl] = zero

    ones = jnp.ones((_L,), jnp.float32)
    cp1.wait()
    cp2.wait()
    cp3.wait()

    for i in range(_CH):
        sl = pl.ds(i * _L, _L)
        lab = lab_v[sl]
        fxc = fx_v[sl]
        fyc = fy_v[sl]
        cxg = plsc.load_gather(cx_v, [lab])
        cyg = plsc.load_gather(cy_v, [lab])
        dx = fxc - cxg
        dy = fyc - cyg
        res_v[sl] = dx * dx + dy * dy
        slot = lab * _L + lane
        plsc.addupdate_scatter(cnt_t, [slot], ones)
        plsc.addupdate_scatter(sx_t, [slot], fxc)
        plsc.addupdate_scatter(sy_t, [slot], fyc)

    pltpu.sync_copy(res_v, res_h.at[pl.ds(base, _SPW)])

    cntv = zero
    sxv = zero
    syv = zero
    for c in range(_NUM_CLASSES):
        m = lane == c
        sl = pl.ds(c * _L, _L)
        cntv = jnp.where(m, jnp.sum(cnt_t[sl]), cntv)
        sxv = jnp.where(m, jnp.sum(sx_t[sl]), sxv)
        syv = jnp.where(m, jnp.sum(sy_t[sl]), syv)
    part_v[pl.ds(0, _L)] = cntv
    part_v[pl.ds(_L, _L)] = sxv
    part_v[pl.ds(2 * _L, _L)] = syv
    pltpu.sync_copy(part_v, shared.at[pl.ds(sid * 3 * _L, 3 * _L)])

    plsc.subcore_barrier()

    @pl.when(sid == 0)
    def _finalize():
        pltpu.sync_copy(shared, rb_v)
        cnt = jnp.zeros((_L,), jnp.float32)
        sx = jnp.zeros((_L,), jnp.float32)
        sy = jnp.zeros((_L,), jnp.float32)
        for w in range(_NW):
            cnt = cnt + rb_v[pl.ds(w * 3 * _L, _L)]
            sx = sx + rb_v[pl.ds(w * 3 * _L + _L, _L)]
            sy = sy + rb_v[pl.ds(w * 3 * _L + 2 * _L, _L)]
        cxr = cx_v[...]
        cyr = cy_v[...]
        scale = _ALPHA / (1.0 + cnt)
        plsc.store_scatter(nc_v, [lane2],
                           cxr - scale * (cnt * cxr - sx), mask=cmask)
        plsc.store_scatter(nc_v, [lane2 + 1],
                           cyr - scale * (cnt * cyr - sy), mask=cmask)
        pltpu.sync_copy(nc_v, nc_h)


_sc_call = functools.partial(
    pl.kernel,
    out_type=(
        jax.ShapeDtypeStruct((_BATCH,), jnp.float32),
        jax.ShapeDtypeStruct((_CLEN,), jnp.float32),
    ),
    mesh=plsc.VectorSubcoreMesh(
        core_axis_name="c", subcore_axis_name="s", num_cores=1
    ),
    compiler_params=pltpu.CompilerParams(needs_layout_passes=False),
    scratch_types=(
        pltpu.VMEM((_SPW,), jnp.float32),
        pltpu.VMEM((_SPW,), jnp.float32),
        pltpu.VMEM((_SPW,), jnp.int32),
        pltpu.VMEM((_SPW,), jnp.float32),
        pltpu.VMEM((_CLEN,), jnp.float32),
        pltpu.VMEM((_L,), jnp.float32),
        pltpu.VMEM((_L,), jnp.float32),
        pltpu.VMEM((_TAB,), jnp.float32),
        pltpu.VMEM((_TAB,), jnp.float32),
        pltpu.VMEM((_TAB,), jnp.float32),
        pltpu.VMEM((3 * _L,), jnp.float32),
        pltpu.VMEM((_NW * 3 * _L,), jnp.float32),
        pltpu.VMEM((_CLEN,), jnp.float32),
        pltpu.VMEM_SHARED((_NW * 3 * _L,), jnp.float32),
        pltpu.SemaphoreType.DMA,
    ),
)(_sc_body)


def kernel(features, labels, centers):
    res, nc = _sc_call(
        features[:, 0], features[:, 1], labels.reshape(-1),
        centers.reshape(-1)
    )
    return (res.reshape(_BATCH, 1), nc.reshape(_NUM_CLASSES, _FEAT_DIM))


# speedup vs baseline: 1.5348x; 1.0835x over previous
import functools

import jax
import jax.numpy as jnp
from jax import lax
from jax.experimental import pallas as pl
from jax.experimental.pallas import tpu as pltpu
from jax.experimental.pallas import tpu_sc as plsc

_NUM_CLASSES = 10
_FEAT_DIM = 2
_ALPHA = 0.5
_BATCH = 16384
_L = 16
_NW = 16
_SPW = _BATCH // _NW
_CH = _SPW // _L
_TAB = _NUM_CLASSES * _L
_CLEN = _NUM_CLASSES * _FEAT_DIM


def _sc_body(fx_h, fy_h, lab_h, c_h,
             res_h, nc_h,
             fx_v, fy_v, lab_v, res_v, c_v, cx_v, cy_v,
             cnt_t, sx_t, sy_t, part_v, rb_v, nc_v,
             shared,
             sem):
    sid = lax.axis_index("s")
    base = sid * _SPW

    cp1 = pltpu.async_copy(fx_h.at[pl.ds(base, _SPW)], fx_v, sem)
    cp2 = pltpu.async_copy(fy_h.at[pl.ds(base, _SPW)], fy_v, sem)
    cp3 = pltpu.async_copy(lab_h.at[pl.ds(base, _SPW)], lab_v, sem)
    pltpu.sync_copy(c_h, c_v)

    lane = lax.iota(jnp.int32, _L)
    lane2 = lane * 2
    cmask = lane < _NUM_CLASSES
    cx_v[...] = plsc.load_gather(c_v, [lane2], mask=cmask)
    cy_v[...] = plsc.load_gather(c_v, [lane2 + 1], mask=cmask)

    zero = jnp.zeros((_L,), jnp.float32)
    for c in range(_NUM_CLASSES):
        sl = pl.ds(c * _L, _L)
        cnt_t[sl] = zero
        sx_t[sl] = zero
        sy_t[sl] = zero

    ones = jnp.ones((_L,), jnp.float32)
    cp1.wait()
    cp2.wait()
    cp3.wait()

    def _chunk(i, _):
        sl = pl.ds(i * _L, _L)
        lab = lab_v[sl]
        fxc = fx_v[sl]
        fyc = fy_v[sl]
        cxg = plsc.load_gather(cx_v, [lab])
        cyg = plsc.load_gather(cy_v, [lab])
        dx = fxc - cxg
        dy = fyc - cyg
        res_v[sl] = dx * dx + dy * dy
        slot = lab * _L + lane
        plsc.addupdate_scatter(cnt_t, [slot], ones)
        plsc.addupdate_scatter(sx_t, [slot], fxc)
        plsc.addupdate_scatter(sy_t, [slot], fyc)
        return 0

    lax.fori_loop(0, _CH, _chunk, 0)

    pltpu.sync_copy(res_v, res_h.at[pl.ds(base, _SPW)])

    def _trcls(c, acc):
        cntv, sxv, syv = acc
        m = lane == c
        sl = pl.ds(c * _L, _L)
        cntv = jnp.where(m, jnp.sum(cnt_t[sl]), cntv)
        sxv = jnp.where(m, jnp.sum(sx_t[sl]), sxv)
        syv = jnp.where(m, jnp.sum(sy_t[sl]), syv)
        return (cntv, sxv, syv)

    cntv, sxv, syv = lax.fori_loop(0, _NUM_CLASSES, _trcls, (zero, zero, zero))
    part_v[pl.ds(0, _L)] = cntv
    part_v[pl.ds(_L, _L)] = sxv
    part_v[pl.ds(2 * _L, _L)] = syv
    pltpu.sync_copy(part_v, shared.at[pl.ds(sid * 3 * _L, 3 * _L)])

    plsc.subcore_barrier()

    @pl.when(sid == 0)
    def _finalize():
        pltpu.sync_copy(shared, rb_v)

        def _wred(w, acc):
            cnt, sx, sy = acc
            return (cnt + rb_v[pl.ds(w * 3 * _L, _L)],
                    sx + rb_v[pl.ds(w * 3 * _L + _L, _L)],
                    sy + rb_v[pl.ds(w * 3 * _L + 2 * _L, _L)])

        z = jnp.zeros((_L,), jnp.float32)
        cnt, sx, sy = lax.fori_loop(0, _NW, _wred, (z, z, z))
        cxr = cx_v[...]
        cyr = cy_v[...]
        scale = _ALPHA / (1.0 + cnt)
        plsc.store_scatter(nc_v, [lane2],
                           cxr - scale * (cnt * cxr - sx), mask=cmask)
        plsc.store_scatter(nc_v, [lane2 + 1],
                           cyr - scale * (cnt * cyr - sy), mask=cmask)
        pltpu.sync_copy(nc_v, nc_h)


_sc_call = functools.partial(
    pl.kernel,
    out_type=(
        jax.ShapeDtypeStruct((_BATCH,), jnp.float32),
        jax.ShapeDtypeStruct((_CLEN,), jnp.float32),
    ),
    mesh=plsc.VectorSubcoreMesh(
        core_axis_name="c", subcore_axis_name="s", num_cores=1
    ),
    compiler_params=pltpu.CompilerParams(needs_layout_passes=False),
    scratch_types=(
        pltpu.VMEM((_SPW,), jnp.float32),
        pltpu.VMEM((_SPW,), jnp.float32),
        pltpu.VMEM((_SPW,), jnp.int32),
        pltpu.VMEM((_SPW,), jnp.float32),
        pltpu.VMEM((_CLEN,), jnp.float32),
        pltpu.VMEM((_L,), jnp.float32),
        pltpu.VMEM((_L,), jnp.float32),
        pltpu.VMEM((_TAB,), jnp.float32),
        pltpu.VMEM((_TAB,), jnp.float32),
        pltpu.VMEM((_TAB,), jnp.float32),
        pltpu.VMEM((3 * _L,), jnp.float32),
        pltpu.VMEM((_NW * 3 * _L,), jnp.float32),
        pltpu.VMEM((_CLEN,), jnp.float32),
        pltpu.VMEM_SHARED((_NW * 3 * _L,), jnp.float32),
        pltpu.SemaphoreType.DMA,
    ),
)(_sc_body)


def kernel(features, labels, centers):
    res, nc = _sc_call(
        features[:, 0], features[:, 1], labels.reshape(-1),
        centers.reshape(-1)
    )
    return (res.reshape(_BATCH, 1), nc.reshape(_NUM_CLASSES, _FEAT_DIM))


# speedup vs baseline: 1.5703x; 1.0231x over previous
import functools

import jax
import jax.numpy as jnp
from jax import lax
from jax.experimental import pallas as pl
from jax.experimental.pallas import tpu as pltpu
from jax.experimental.pallas import tpu_sc as plsc

_NUM_CLASSES = 10
_FEAT_DIM = 2
_ALPHA = 0.5
_BATCH = 16384
_L = 16
_NW = 16
_SPW = _BATCH // _NW
_CH = _SPW // _L
_TAB = _NUM_CLASSES * _L
_CLEN = _NUM_CLASSES * _FEAT_DIM


def _sc_body(fx_h, fy_h, lab_h, c_h,
             res_h, nc_h,
             fx_v, fy_v, lab_v, res_v, c_v, cx_v, cy_v,
             cnt_t, sx_t, sy_t, part_v, rb_v, nc_v,
             shared,
             sem):
    sid = lax.axis_index("s")
    base = sid * _SPW

    cp1 = pltpu.async_copy(fx_h.at[pl.ds(base, _SPW)], fx_v, sem)
    cp2 = pltpu.async_copy(fy_h.at[pl.ds(base, _SPW)], fy_v, sem)
    cp3 = pltpu.async_copy(lab_h.at[pl.ds(base, _SPW)], lab_v, sem)
    pltpu.sync_copy(c_h, c_v)

    lane = lax.iota(jnp.int32, _L)
    lane2 = lane * 2
    cmask = lane < _NUM_CLASSES
    cx_v[...] = plsc.load_gather(c_v, [lane2], mask=cmask)
    cy_v[...] = plsc.load_gather(c_v, [lane2 + 1], mask=cmask)

    zero = jnp.zeros((_L,), jnp.float32)
    for c in range(_NUM_CLASSES):
        sl = pl.ds(c * _L, _L)
        cnt_t[sl] = zero
        sx_t[sl] = zero
        sy_t[sl] = zero

    ones = jnp.ones((_L,), jnp.float32)
    cp1.wait()
    cp2.wait()
    cp3.wait()

    @plsc.parallel_loop(0, _SPW, _L, unroll=4)
    def _chunk(off):
        sl = pl.ds(off, _L)
        lab = lab_v[sl]
        fxc = fx_v[sl]
        fyc = fy_v[sl]
        cxg = plsc.load_gather(cx_v, [lab])
        cyg = plsc.load_gather(cy_v, [lab])
        dx = fxc - cxg
        dy = fyc - cyg
        res_v[sl] = dx * dx + dy * dy
        slot = lab * _L + lane
        plsc.addupdate_scatter(cnt_t, [slot], ones)
        plsc.addupdate_scatter(sx_t, [slot], fxc)
        plsc.addupdate_scatter(sy_t, [slot], fyc)

    pltpu.sync_copy(res_v, res_h.at[pl.ds(base, _SPW)])

    def _trcls(c, acc):
        cntv, sxv, syv = acc
        m = lane == c
        sl = pl.ds(c * _L, _L)
        cntv = jnp.where(m, jnp.sum(cnt_t[sl]), cntv)
        sxv = jnp.where(m, jnp.sum(sx_t[sl]), sxv)
        syv = jnp.where(m, jnp.sum(sy_t[sl]), syv)
        return (cntv, sxv, syv)

    cntv, sxv, syv = lax.fori_loop(0, _NUM_CLASSES, _trcls, (zero, zero, zero))
    part_v[pl.ds(0, _L)] = cntv
    part_v[pl.ds(_L, _L)] = sxv
    part_v[pl.ds(2 * _L, _L)] = syv
    pltpu.sync_copy(part_v, shared.at[pl.ds(sid * 3 * _L, 3 * _L)])

    plsc.subcore_barrier()

    @pl.when(sid == 0)
    def _finalize():
        pltpu.sync_copy(shared, rb_v)

        def _wred(w, acc):
            cnt, sx, sy = acc
            return (cnt + rb_v[pl.ds(w * 3 * _L, _L)],
                    sx + rb_v[pl.ds(w * 3 * _L + _L, _L)],
                    sy + rb_v[pl.ds(w * 3 * _L + 2 * _L, _L)])

        z = jnp.zeros((_L,), jnp.float32)
        cnt, sx, sy = lax.fori_loop(0, _NW, _wred, (z, z, z))
        cxr = cx_v[...]
        cyr = cy_v[...]
        scale = _ALPHA / (1.0 + cnt)
        plsc.store_scatter(nc_v, [lane2],
                           cxr - scale * (cnt * cxr - sx), mask=cmask)
        plsc.store_scatter(nc_v, [lane2 + 1],
                           cyr - scale * (cnt * cyr - sy), mask=cmask)
        pltpu.sync_copy(nc_v, nc_h)


_sc_call = functools.partial(
    pl.kernel,
    out_type=(
        jax.ShapeDtypeStruct((_BATCH,), jnp.float32),
        jax.ShapeDtypeStruct((_CLEN,), jnp.float32),
    ),
    mesh=plsc.VectorSubcoreMesh(
        core_axis_name="c", subcore_axis_name="s", num_cores=1
    ),
    compiler_params=pltpu.CompilerParams(needs_layout_passes=False),
    scratch_types=(
        pltpu.VMEM((_SPW,), jnp.float32),
        pltpu.VMEM((_SPW,), jnp.float32),
        pltpu.VMEM((_SPW,), jnp.int32),
        pltpu.VMEM((_SPW,), jnp.float32),
        pltpu.VMEM((_CLEN,), jnp.float32),
        pltpu.VMEM((_L,), jnp.float32),
        pltpu.VMEM((_L,), jnp.float32),
        pltpu.VMEM((_TAB,), jnp.float32),
        pltpu.VMEM((_TAB,), jnp.float32),
        pltpu.VMEM((_TAB,), jnp.float32),
        pltpu.VMEM((3 * _L,), jnp.float32),
        pltpu.VMEM((_NW * 3 * _L,), jnp.float32),
        pltpu.VMEM((_CLEN,), jnp.float32),
        pltpu.VMEM_SHARED((_NW * 3 * _L,), jnp.float32),
        pltpu.SemaphoreType.DMA,
    ),
)(_sc_body)


def kernel(features, labels, centers):
    res, nc = _sc_call(
        features[:, 0], features[:, 1], labels.reshape(-1),
        centers.reshape(-1)
    )
    return (res.reshape(_BATCH, 1), nc.reshape(_NUM_CLASSES, _FEAT_DIM))
